# baseline (speedup 1.0000x reference)
import jax
import jax.numpy as jnp
from jax import lax
from jax.experimental import pallas as pl
from jax.experimental.pallas import tpu as pltpu
from jax.experimental.pallas import tpu_sc as plsc

MAXN = 100000
SEQL = 512
NC = 2
NS = 16
NW = NC * NS
CH = 112
NCHUNKS = MAXN // CH
TAIL0 = NCHUNKS * CH
TAILR = MAXN - TAIL0
NPW = -(-NCHUNKS // NW)
SEG = 16
PIECE = 5000
W_LEN = 28
W_LP = 29
W_SZ = 30
W_TL = 31


def _body(seq_in, len_in, lp_in, sc_in, nseq_in,
          seq_out, len_out, lp_out, size_out,
          sc_v, seg_v, blk_v, pc_v, sp0, sp1, spt,
          gsem0, gsem1, ssem0, ssem1):
    wid = lax.axis_index("s") * NC + lax.axis_index("c")
    sid = lax.axis_index("s")
    sps = (sp0, sp1)
    gsems = (gsem0, gsem1)
    ssems = (ssem0, ssem1)

    pltpu.sync_copy(sc_in, sc_v)
    sc_vec = sc_v[...]
    s = sc_vec[0]
    nlen = sc_vec[1]
    nlp_bits = sc_vec[2]

    sd = [None] * NPW
    for j in range(NPW):
        b = j % 2
        cid = j * NW + wid
        r = cid * CH

        if j >= 2:
            @pl.when((j - 2) * NW + wid < NCHUNKS)
            def _():
                sd[j - 2].wait()

        @pl.when(cid < NCHUNKS)
        def _():
            gd = pltpu.async_copy(seq_in.at[pl.ds(r, CH)], sps[b].at[sid],
                                  gsems[b])
            gd.wait()
            sd[j] = pltpu.async_copy(sps[b].at[sid], seq_out.at[pl.ds(r, CH)],
                                     ssems[b])

    for j in range(max(0, NPW - 2), NPW):
        @pl.when(j * NW + wid < NCHUNKS)
        def _():
            sd[j].wait()

    @pl.when(wid == W_TL)
    def _():
        pltpu.sync_copy(seq_in.at[pl.ds(TAIL0, TAILR)], spt)
        pltpu.sync_copy(spt, seq_out.at[pl.ds(TAIL0, TAILR)])

    cs = s // CH
    owner = jnp.where(cs >= NCHUNKS, W_TL, cs % NW)

    @pl.when((s < MAXN) & (wid == owner))
    def _():
        rb = (s // 8) * 8
        pltpu.sync_copy(seq_in.at[pl.ds(rb, 8)], blk_v)
        pltpu.sync_copy(nseq_in, blk_v.at[s - rb])
        pltpu.sync_copy(blk_v, seq_out.at[pl.ds(rb, 8)])

    lane = lax.iota(jnp.int32, SEG)
    b16 = (s // SEG) * SEG
    c = s - b16

    @pl.when(wid == W_LEN)
    def _():
        for t in range(MAXN // PIECE):
            pltpu.sync_copy(len_in.at[pl.ds(t * PIECE, PIECE)], pc_v)
            pltpu.sync_copy(pc_v, len_out.at[pl.ds(t * PIECE, PIECE)])

        @pl.when(s < MAXN)
        def _():
            pltpu.sync_copy(len_in.at[pl.ds(b16, SEG)], seg_v)
            seg_v[...] = jnp.where(lane == c, nlen, seg_v[...])
            pltpu.sync_copy(seg_v, len_out.at[pl.ds(b16, SEG)])

    @pl.when(wid == W_LP)
    def _():
        for t in range(MAXN // PIECE):
            pltpu.sync_copy(lp_in.at[pl.ds(t * PIECE, PIECE)], pc_v)
            pltpu.sync_copy(pc_v, lp_out.at[pl.ds(t * PIECE, PIECE)])

        @pl.when(s < MAXN)
        def _():
            pltpu.sync_copy(lp_in.at[pl.ds(b16, SEG)], seg_v)
            seg_v[...] = jnp.where(lane == c, nlp_bits, seg_v[...])
            pltpu.sync_copy(seg_v, lp_out.at[pl.ds(b16, SEG)])

    @pl.when(wid == W_SZ)
    def _():
        seg_v[...] = jnp.where(lane == 0, s + 1, 0)
        pltpu.sync_copy(seg_v, size_out)


_tree_add = pl.kernel(
    _body,
    out_type=(
        jax.ShapeDtypeStruct((MAXN, SEQL), jnp.int32),
        jax.ShapeDtypeStruct((MAXN,), jnp.int32),
        jax.ShapeDtypeStruct((MAXN,), jnp.int32),
        jax.ShapeDtypeStruct((SEG,), jnp.int32),
    ),
    mesh=plsc.VectorSubcoreMesh(core_axis_name="c", subcore_axis_name="s"),
    scratch_types=[
        pltpu.VMEM((SEG,), jnp.int32),
        pltpu.VMEM((SEG,), jnp.int32),
        pltpu.VMEM((8, SEQL), jnp.int32),
        pltpu.VMEM((PIECE,), jnp.int32),
        pltpu.VMEM_SHARED((NS, CH, SEQL), jnp.int32),
        pltpu.VMEM_SHARED((NS, CH, SEQL), jnp.int32),
        pltpu.VMEM_SHARED((TAILR, SEQL), jnp.int32),
        pltpu.SemaphoreType.DMA,
        pltpu.SemaphoreType.DMA,
        pltpu.SemaphoreType.DMA,
        pltpu.SemaphoreType.DMA,
    ],
)


def kernel(sequences, sequence_lengths, log_probabilities, size,
           node_sequence, node_sequence_length, node_log_probability):
    lp_bits = lax.bitcast_convert_type(node_log_probability, jnp.int32)
    scalars = (jnp.zeros((SEG,), jnp.int32)
               .at[0].set(size)
               .at[1].set(node_sequence_length)
               .at[2].set(lp_bits))
    lp_in = lax.bitcast_convert_type(log_probabilities, jnp.int32)
    seq_o, len_o, lp_o, size_o = _tree_add(
        sequences, sequence_lengths, lp_in, scalars, node_sequence)
    return (seq_o, len_o,
            lax.bitcast_convert_type(lp_o, jnp.float32),
            size_o[0])


# speedup vs baseline: 1.0570x; 1.0570x over previous
import jax
import jax.numpy as jnp
from jax import lax
from jax.experimental import pallas as pl
from jax.experimental.pallas import tpu as pltpu
from jax.experimental.pallas import tpu_sc as plsc

MAXN = 100000
SEQL = 512
NC = 2
NS = 16
NW = NC * NS
CH = 80
NCHUNKS = MAXN // CH
NPW = -(-NCHUNKS // NW)
NIT = NPW // 2
SEG = 16
PIECE = 5000


def _body(seq_in, len_in, lp_in, sc_in, nseq_in,
          seq_out, len_out, lp_out, size_out,
          sc_v, seg_v, pc_v, buf0, buf1, sp_v,
          gsem0, gsem1, ssem0, ssem1):
    wid = lax.axis_index("s") * NC + lax.axis_index("c")
    sid = lax.axis_index("s")
    bufs = (buf0, buf1)
    gsems = (gsem0, gsem1)
    ssems = (ssem0, ssem1)

    pltpu.sync_copy(sc_in, sc_v)
    sc_vec = sc_v[...]
    s = sc_vec[0]
    nlen = sc_vec[1]
    nlp_bits = sc_vec[2]

    base = wid * NPW
    sds = [None] * NIT
    for i in range(NIT):
        b = i % 2
        cid0 = base + 2 * i
        cid1 = cid0 + 1
        r0 = cid0 * CH
        r1 = cid1 * CH

        if i >= 2:
            @pl.when(base + 2 * (i - 2) + 1 < NCHUNKS)
            def _():
                sds[i - 2].wait()

        @pl.when(cid1 < NCHUNKS)
        def _():
            gds = pltpu.async_copy(seq_in.at[pl.ds(r1, CH)], bufs[b],
                                   gsems[b])
            del gds

        @pl.when(cid0 < NCHUNKS)
        def _():
            pltpu.sync_copy(seq_in.at[pl.ds(r0, CH)], sp_v.at[sid])
            pltpu.sync_copy(sp_v.at[sid], seq_out.at[pl.ds(r0, CH)])

        @pl.when(cid1 < NCHUNKS)
        def _():
            pltpu.make_async_copy(seq_in.at[pl.ds(r1, CH)], bufs[b],
                                  gsems[b]).wait()
            sds[i] = pltpu.async_copy(bufs[b], seq_out.at[pl.ds(r1, CH)],
                                      ssems[b])

    for i in range(max(0, NIT - 2), NIT):
        @pl.when(base + 2 * i + 1 < NCHUNKS)
        def _():
            sds[i].wait()

    cs = s // CH

    @pl.when((s < MAXN) & (cs >= base) & (cs < base + NPW))
    def _():
        rb = (s // 8) * 8
        blk = buf0.at[pl.ds(0, 8)]
        pltpu.sync_copy(seq_in.at[pl.ds(rb, 8)], blk)
        pltpu.sync_copy(nseq_in, buf0.at[s - rb])
        pltpu.sync_copy(blk, seq_out.at[pl.ds(rb, 8)])

    lane = lax.iota(jnp.int32, SEG)
    b16 = (s // SEG) * SEG
    c = s - b16

    @pl.when(wid == NW - 1)
    def _():
        for t in range(MAXN // PIECE):
            pltpu.sync_copy(len_in.at[pl.ds(t * PIECE, PIECE)], pc_v)
            pltpu.sync_copy(pc_v, len_out.at[pl.ds(t * PIECE, PIECE)])
            pltpu.sync_copy(lp_in.at[pl.ds(t * PIECE, PIECE)], pc_v)
            pltpu.sync_copy(pc_v, lp_out.at[pl.ds(t * PIECE, PIECE)])

        @pl.when(s < MAXN)
        def _():
            pltpu.sync_copy(len_in.at[pl.ds(b16, SEG)], seg_v)
            seg_v[...] = jnp.where(lane == c, nlen, seg_v[...])
            pltpu.sync_copy(seg_v, len_out.at[pl.ds(b16, SEG)])
            pltpu.sync_copy(lp_in.at[pl.ds(b16, SEG)], seg_v)
            seg_v[...] = jnp.where(lane == c, nlp_bits, seg_v[...])
            pltpu.sync_copy(seg_v, lp_out.at[pl.ds(b16, SEG)])

        seg_v[...] = jnp.where(lane == 0, s + 1, 0)
        pltpu.sync_copy(seg_v, size_out)


_tree_add = pl.kernel(
    _body,
    out_type=(
        jax.ShapeDtypeStruct((MAXN, SEQL), jnp.int32),
        jax.ShapeDtypeStruct((MAXN,), jnp.int32),
        jax.ShapeDtypeStruct((MAXN,), jnp.int32),
        jax.ShapeDtypeStruct((SEG,), jnp.int32),
    ),
    mesh=plsc.VectorSubcoreMesh(core_axis_name="c", subcore_axis_name="s"),
    scratch_types=[
        pltpu.VMEM((SEG,), jnp.int32),
        pltpu.VMEM((SEG,), jnp.int32),
        pltpu.VMEM((PIECE,), jnp.int32),
        pltpu.VMEM((CH, SEQL), jnp.int32),
        pltpu.VMEM((CH, SEQL), jnp.int32),
        pltpu.VMEM_SHARED((NS, CH, SEQL), jnp.int32),
        pltpu.SemaphoreType.DMA,
        pltpu.SemaphoreType.DMA,
        pltpu.SemaphoreType.DMA,
        pltpu.SemaphoreType.DMA,
    ],
)


def kernel(sequences, sequence_lengths, log_probabilities, size,
           node_sequence, node_sequence_length, node_log_probability):
    lp_bits = lax.bitcast_convert_type(node_log_probability, jnp.int32)
    scalars = (jnp.zeros((SEG,), jnp.int32)
               .at[0].set(size)
               .at[1].set(node_sequence_length)
               .at[2].set(lp_bits))
    lp_in = lax.bitcast_convert_type(log_probabilities, jnp.int32)
    seq_o, len_o, lp_o, size_o = _tree_add(
        sequences, sequence_lengths, lp_in, scalars, node_sequence)
    return (seq_o, len_o,
            lax.bitcast_convert_type(lp_o, jnp.float32),
            size_o[0])


# speedup vs baseline: 1.1304x; 1.0694x over previous
import jax
import jax.numpy as jnp
from jax import lax
from jax.experimental import pallas as pl
from jax.experimental.pallas import tpu as pltpu
from jax.experimental.pallas import tpu_sc as plsc

MAXN = 100000
SEQL = 512
BLK = 2000
NC = 2
NS = 16
SEG = 16
SLAB1D = 4000
NW1D = MAXN // SLAB1D


def _tc_body(size_ref, seq_ref, nseq_ref, out_ref):
    i = pl.program_id(0)
    local = size_ref[0] - i * BLK
    rows = lax.broadcasted_iota(jnp.int32, (BLK, 1), 0)
    out_ref[...] = jnp.where(rows == local, nseq_ref[...], seq_ref[...])


def _tc_copy(size1, sequences, nseq2d):
    return pl.pallas_call(
        _tc_body,
        grid=(MAXN // BLK,),
        in_specs=[
            pl.BlockSpec(memory_space=pltpu.SMEM),
            pl.BlockSpec((BLK, SEQL), lambda i: (i, 0)),
            pl.BlockSpec((1, SEQL), lambda i: (0, 0)),
        ],
        out_specs=pl.BlockSpec((BLK, SEQL), lambda i: (i, 0)),
        out_shape=jax.ShapeDtypeStruct((MAXN, SEQL), jnp.int32),
    )(size1, sequences, nseq2d)


def _sc_body(len_in, lp_in, sc_in,
             len_out, lp_out, size_out,
             sc_v, seg_v, sl_i, sl_f):
    wid = lax.axis_index("s") * NC + lax.axis_index("c")
    pltpu.sync_copy(sc_in, sc_v)
    sc_vec = sc_v[...]
    s = sc_vec[0]
    nlen = sc_vec[1]
    nlp_bits = sc_vec[2]

    @pl.when(wid < NW1D)
    def _():
        o = wid * SLAB1D
        pltpu.sync_copy(len_in.at[pl.ds(o, SLAB1D)], sl_i)
        pltpu.sync_copy(sl_i, len_out.at[pl.ds(o, SLAB1D)])
        pltpu.sync_copy(lp_in.at[pl.ds(o, SLAB1D)], sl_f)
        pltpu.sync_copy(sl_f, lp_out.at[pl.ds(o, SLAB1D)])

    lane = lax.iota(jnp.int32, SEG)
    b16 = (s // SEG) * SEG
    c = s - b16

    @pl.when((s < MAXN) & (wid == s // SLAB1D))
    def _():
        pltpu.sync_copy(len_in.at[pl.ds(b16, SEG)], seg_v)
        seg_v[...] = jnp.where(lane == c, nlen, seg_v[...])
        pltpu.sync_copy(seg_v, len_out.at[pl.ds(b16, SEG)])
        pltpu.sync_copy(lp_in.at[pl.ds(b16, SEG)], seg_v)
        seg_v[...] = jnp.where(lane == c, nlp_bits, seg_v[...])
        pltpu.sync_copy(seg_v, lp_out.at[pl.ds(b16, SEG)])

    @pl.when(wid == NW1D)
    def _():
        seg_v[...] = jnp.where(lane == 0, s + 1, 0)
        pltpu.sync_copy(seg_v, size_out)


_sc_small = pl.kernel(
    _sc_body,
    out_type=(
        jax.ShapeDtypeStruct((MAXN,), jnp.int32),
        jax.ShapeDtypeStruct((MAXN,), jnp.int32),
        jax.ShapeDtypeStruct((SEG,), jnp.int32),
    ),
    mesh=plsc.VectorSubcoreMesh(core_axis_name="c", subcore_axis_name="s"),
    scratch_types=[
        pltpu.VMEM((SEG,), jnp.int32),
        pltpu.VMEM((SEG,), jnp.int32),
        pltpu.VMEM((SLAB1D,), jnp.int32),
        pltpu.VMEM((SLAB1D,), jnp.int32),
    ],
)


def kernel(sequences, sequence_lengths, log_probabilities, size,
           node_sequence, node_sequence_length, node_log_probability):
    lp_bits = lax.bitcast_convert_type(node_log_probability, jnp.int32)
    scalars = (jnp.zeros((SEG,), jnp.int32)
               .at[0].set(size)
               .at[1].set(node_sequence_length)
               .at[2].set(lp_bits))
    lp_in = lax.bitcast_convert_type(log_probabilities, jnp.int32)
    len_o, lp_o, size_o = _sc_small(sequence_lengths, lp_in, scalars)
    seq_o = _tc_copy(size.reshape(1), sequences,
                     node_sequence.reshape(1, SEQL))
    return (seq_o, len_o,
            lax.bitcast_convert_type(lp_o, jnp.float32),
            size_o[0])
